# baseline (speedup 1.0000x reference)
import functools

import jax
import jax.numpy as jnp
from jax import lax
from jax.experimental import pallas as pl
from jax.experimental.pallas import tpu as pltpu
from jax.experimental.pallas import tpu_sc as plsc

_L = 32
_H = 128
_R = 11
_IN = 128
_OUT = 128

_N_WORKERS = 32

_SPLITS = 2

_BLK = 1000


def _sc_gather(table, idx_flat, chunk, sub):
    total = idx_flat.shape[0]
    d = table.shape[1]
    per_w = total // _N_WORKERS
    n_chunks = per_w // chunk
    n_pairs = (n_chunks - 1) // 2
    mesh = plsc.VectorSubcoreMesh(core_axis_name="c", subcore_axis_name="s")

    @functools.partial(
        pl.kernel,
        mesh=mesh,
        out_type=jax.ShapeDtypeStruct((total, d), jnp.float32),
        scratch_types=[
            pltpu.VMEM((per_w,), jnp.int32),
            pltpu.VMEM((chunk, d), jnp.float32),
            pltpu.VMEM((chunk, d), jnp.float32),
            pltpu.SemaphoreType.DMA,
            pltpu.SemaphoreType.DMA,
            pltpu.SemaphoreType.DMA,
            pltpu.SemaphoreType.DMA,
        ],
    )
    def gather_kernel(table_hbm, idx_hbm, out_hbm, idx_all, rows0, rows1,
                      sg0, sg1, ss0, ss1):
        wid = lax.axis_index("s") * 2 + lax.axis_index("c")
        base = wid * per_w
        pltpu.sync_copy(idx_hbm.at[pl.ds(base, per_w)], idx_all)

        def fire_gather(c, rows_buf, sem):
            for k in range(chunk // sub):
                pltpu.async_copy(
                    table_hbm.at[idx_all.at[pl.ds(c * chunk + k * sub, sub)]],
                    rows_buf.at[pl.ds(k * sub, sub)],
                    sem,
                )

        def drain_gather(rows_buf, sem):
            pltpu.make_async_copy(out_hbm.at[pl.ds(base, chunk)], rows_buf, sem).wait()

        def fire_store(c, rows_buf, sem):
            pltpu.async_copy(rows_buf, out_hbm.at[pl.ds(base + c * chunk, chunk)], sem)

        def drain_store(rows_buf, sem):
            pltpu.make_async_copy(rows_buf, out_hbm.at[pl.ds(base, chunk)], sem).wait()

        fire_gather(0, rows0, sg0)

        def body(p, carry):
            a = 2 * p + 1
            b = 2 * p + 2

            @pl.when(p > 0)
            def _():
                drain_store(rows1, ss1)
            fire_gather(a, rows1, sg1)
            drain_gather(rows0, sg0)
            fire_store(2 * p, rows0, ss0)
            drain_store(rows0, ss0)
            fire_gather(b, rows0, sg0)
            drain_gather(rows1, sg1)
            fire_store(a, rows1, ss1)
            return carry

        lax.fori_loop(0, n_pairs, body, 0)

        drain_gather(rows0, sg0)
        fire_store(n_chunks - 1, rows0, ss0)
        drain_store(rows0, ss0)
        drain_store(rows1, ss1)

    return gather_kernel(table, idx_flat)


def _tc_body(xg_ref, rels_ref, adj_ref, wih_ref, whh_ref, wflat_ref, out_ref,
             h_ref, c_ref, s_ref):
    t = pl.program_id(1)

    @pl.when(t == 0)
    def _init():
        h_ref[...] = jnp.zeros_like(h_ref)
        c_ref[...] = jnp.zeros_like(c_ref)
        s_ref[...] = jnp.zeros_like(s_ref)

    x_t = xg_ref[...].astype(jnp.bfloat16)
    gates = jnp.dot(x_t, wih_ref[...], preferred_element_type=jnp.float32)
    gates = gates + jnp.dot(h_ref[...].astype(jnp.bfloat16), whh_ref[...],
                            preferred_element_type=jnp.float32)
    i_g = jax.nn.sigmoid(gates[:, 0:_H])
    f_g = jax.nn.sigmoid(gates[:, _H:2 * _H])
    g_g = jnp.tanh(gates[:, 2 * _H:3 * _H])
    o_g = jax.nn.sigmoid(gates[:, 3 * _H:4 * _H])
    c = f_g * c_ref[...] + i_g * g_g
    h = o_g * jnp.tanh(c)
    c_ref[...] = c
    h_ref[...] = h

    blk = adj_ref.shape[0]
    sel = lax.broadcasted_iota(jnp.int32, (blk, _L), 1) == t
    rel_t = jnp.sum(jnp.where(sel, rels_ref[...], 0), axis=1, keepdims=True)
    adj_t = jnp.sum(jnp.where(sel, adj_ref[...], 0.0), axis=1, keepdims=True)
    for r in range(_R):
        coeff = jnp.where(rel_t == r, adj_t, 0.0)
        s_ref[:, r * _H:(r + 1) * _H] += coeff * h

    @pl.when(t == _L - 1)
    def _finish():
        out_ref[...] = jnp.maximum(
            jnp.dot(s_ref[...], wflat_ref[...], preferred_element_type=jnp.float32),
            0.0,
        )


def _tc_lstm_rgc(xgr, rels, adj, wih_t, whh_t, wflat, interpret=False):
    n = rels.shape[0]
    grid = (n // _BLK, _L)
    return pl.pallas_call(
        _tc_body,
        grid=grid,
        in_specs=[
            pl.BlockSpec((_BLK, _IN), lambda i, t: (i, t)),
            pl.BlockSpec((_BLK, _L), lambda i, t: (i, 0)),
            pl.BlockSpec((_BLK, _L), lambda i, t: (i, 0)),
            pl.BlockSpec((_IN, 4 * _H), lambda i, t: (0, 0)),
            pl.BlockSpec((_H, 4 * _H), lambda i, t: (0, 0)),
            pl.BlockSpec((_R * _H, _OUT), lambda i, t: (0, 0)),
        ],
        out_specs=pl.BlockSpec((_BLK, _OUT), lambda i, t: (i, 0)),
        out_shape=jax.ShapeDtypeStruct((n, _OUT), jnp.float32),
        scratch_shapes=[
            pltpu.VMEM((_BLK, _H), jnp.float32),
            pltpu.VMEM((_BLK, _H), jnp.float32),
            pltpu.VMEM((_BLK, _R * _H), jnp.float32),
        ],
        interpret=interpret,
    )(xgr, rels, adj, wih_t, whh_t, wflat)


def kernel(features, neighbor_ids, neighbor_rels, adj_weights, W_ih, W_hh, rgc_weights):
    n, l = neighbor_ids.shape
    ids = neighbor_ids.astype(jnp.int32)
    rels = neighbor_rels.astype(jnp.int32)
    adj = adj_weights.astype(jnp.float32)
    wih_t = W_ih.T.astype(jnp.bfloat16)
    whh_t = W_hh.T.astype(jnp.bfloat16)
    wflat = rgc_weights.reshape(_R * _H, _OUT)

    seg = n // _SPLITS
    per_w = seg * l // _N_WORKERS
    chunk = 400
    while per_w % chunk or (per_w // chunk) % 2 == 0:
        chunk //= 2
    sub = chunk // 5

    outs = []
    for k in range(_SPLITS):
        sl = slice(k * seg, (k + 1) * seg)
        idx_flat = ids[sl].reshape(-1)
        xg = _sc_gather(features, idx_flat, chunk, sub)
        xgr = xg.reshape(seg, l * _IN)
        outs.append(_tc_lstm_rgc(xgr, rels[sl], adj[sl], wih_t, whh_t, wflat))
    if _SPLITS == 1:
        return outs[0]
    return jnp.concatenate(outs, axis=0)


# speedup vs baseline: 1.1790x; 1.1790x over previous
import functools

import jax
import jax.numpy as jnp
from jax import lax
from jax.experimental import pallas as pl
from jax.experimental.pallas import tpu as pltpu
from jax.experimental.pallas import tpu_sc as plsc

_L = 32
_H = 128
_R = 11
_IN = 128
_OUT = 128

_N_WORKERS = 32

_SPLITS = 2

_BLK = 1000


def _sc_gather(table, idx_flat, chunk, sub):
    total = idx_flat.shape[0]
    d = table.shape[1]
    per_w = total // _N_WORKERS
    n_chunks = per_w // chunk
    n_pairs = (n_chunks - 1) // 2
    mesh = plsc.VectorSubcoreMesh(core_axis_name="c", subcore_axis_name="s")

    @functools.partial(
        pl.kernel,
        mesh=mesh,
        out_type=jax.ShapeDtypeStruct((total, d), jnp.float32),
        scratch_types=[
            pltpu.VMEM((per_w,), jnp.int32),
            pltpu.VMEM((chunk, d), jnp.float32),
            pltpu.VMEM((chunk, d), jnp.float32),
            pltpu.SemaphoreType.DMA,
            pltpu.SemaphoreType.DMA,
            pltpu.SemaphoreType.DMA,
            pltpu.SemaphoreType.DMA,
        ],
    )
    def gather_kernel(table_hbm, idx_hbm, out_hbm, idx_all, rows0, rows1,
                      sg0, sg1, ss0, ss1):
        wid = lax.axis_index("s") * 2 + lax.axis_index("c")
        base = wid * per_w
        pltpu.sync_copy(idx_hbm.at[pl.ds(base, per_w)], idx_all)

        def fire_gather(c, rows_buf, sem):
            for k in range(chunk // sub):
                pltpu.async_copy(
                    table_hbm.at[idx_all.at[pl.ds(c * chunk + k * sub, sub)]],
                    rows_buf.at[pl.ds(k * sub, sub)],
                    sem,
                )

        def drain_gather(rows_buf, sem):
            pltpu.make_async_copy(out_hbm.at[pl.ds(base, chunk)], rows_buf, sem).wait()

        def fire_store(c, rows_buf, sem):
            pltpu.async_copy(rows_buf, out_hbm.at[pl.ds(base + c * chunk, chunk)], sem)

        def drain_store(rows_buf, sem):
            pltpu.make_async_copy(rows_buf, out_hbm.at[pl.ds(base, chunk)], sem).wait()

        fire_gather(0, rows0, sg0)

        def body(p, carry):
            a = 2 * p + 1
            b = 2 * p + 2

            @pl.when(p > 0)
            def _():
                drain_store(rows1, ss1)
            fire_gather(a, rows1, sg1)
            drain_gather(rows0, sg0)
            fire_store(2 * p, rows0, ss0)
            drain_store(rows0, ss0)
            fire_gather(b, rows0, sg0)
            drain_gather(rows1, sg1)
            fire_store(a, rows1, ss1)
            return carry

        lax.fori_loop(0, n_pairs, body, 0)

        drain_gather(rows0, sg0)
        fire_store(n_chunks - 1, rows0, ss0)
        drain_store(rows0, ss0)
        drain_store(rows1, ss1)

    return gather_kernel(table, idx_flat)


_U = 8


def _tc_body(xg_ref, rels_ref, adj_ref, wih_ref, whh_ref, wflat_ref, out_ref,
             h_ref, c_ref, s_ref):
    tb = pl.program_id(1)

    @pl.when(tb == 0)
    def _init():
        h_ref[...] = jnp.zeros_like(h_ref)
        c_ref[...] = jnp.zeros_like(c_ref)
        s_ref[...] = jnp.zeros_like(s_ref)

    blk = adj_ref.shape[0]
    row = lax.broadcasted_iota(jnp.int32, (_L, 128), 0)
    col = lax.broadcasted_iota(jnp.int32, (_L, 128), 1)
    onehot = jnp.where((row == tb * _U + col) & (col < _U), 1.0, 0.0)
    cols_rel = jnp.dot(rels_ref[...], onehot, preferred_element_type=jnp.float32)
    cols_adj = jnp.dot(adj_ref[...], onehot, preferred_element_type=jnp.float32)

    h = h_ref[...]
    c = c_ref[...]
    for u in range(_U):
        x_u = xg_ref[:, u, :]
        gates = jnp.dot(x_u, wih_ref[...], preferred_element_type=jnp.float32)
        gates = gates + jnp.dot(h, whh_ref[...], preferred_element_type=jnp.float32)
        i_g = jax.nn.sigmoid(gates[:, 0:_H])
        f_g = jax.nn.sigmoid(gates[:, _H:2 * _H])
        g_g = jnp.tanh(gates[:, 2 * _H:3 * _H])
        o_g = jax.nn.sigmoid(gates[:, 3 * _H:4 * _H])
        c = f_g * c + i_g * g_g
        h = o_g * jnp.tanh(c)
        rel_u = cols_rel[:, u:u + 1]
        adj_u = cols_adj[:, u:u + 1]
        for r in range(_R):
            coeff = jnp.where(rel_u == r, adj_u, 0.0)
            s_ref[:, r * _H:(r + 1) * _H] += coeff * h
    h_ref[...] = h
    c_ref[...] = c

    @pl.when(tb == _L // _U - 1)
    def _finish():
        out_ref[...] = jnp.maximum(
            jnp.dot(s_ref[...], wflat_ref[...], preferred_element_type=jnp.float32),
            0.0,
        )


def _tc_lstm_rgc(xg3, rels, adj, wih_t, whh_t, wflat, interpret=False):
    n = rels.shape[0]
    grid = (n // _BLK, _L // _U)
    return pl.pallas_call(
        _tc_body,
        grid=grid,
        in_specs=[
            pl.BlockSpec((_BLK, _U, _IN), lambda i, t: (i, t, 0)),
            pl.BlockSpec((_BLK, _L), lambda i, t: (i, 0)),
            pl.BlockSpec((_BLK, _L), lambda i, t: (i, 0)),
            pl.BlockSpec((_IN, 4 * _H), lambda i, t: (0, 0)),
            pl.BlockSpec((_H, 4 * _H), lambda i, t: (0, 0)),
            pl.BlockSpec((_R * _H, _OUT), lambda i, t: (0, 0)),
        ],
        out_specs=pl.BlockSpec((_BLK, _OUT), lambda i, t: (i, 0)),
        out_shape=jax.ShapeDtypeStruct((n, _OUT), jnp.float32),
        scratch_shapes=[
            pltpu.VMEM((_BLK, _H), jnp.float32),
            pltpu.VMEM((_BLK, _H), jnp.float32),
            pltpu.VMEM((_BLK, _R * _H), jnp.float32),
        ],
        interpret=interpret,
    )(xg3, rels, adj, wih_t, whh_t, wflat)


def kernel(features, neighbor_ids, neighbor_rels, adj_weights, W_ih, W_hh, rgc_weights):
    n, l = neighbor_ids.shape
    ids = neighbor_ids.astype(jnp.int32)
    rels_f = neighbor_rels.astype(jnp.float32)
    adj = adj_weights.astype(jnp.float32)
    wih_t = W_ih.T
    whh_t = W_hh.T
    wflat = rgc_weights.reshape(_R * _H, _OUT)

    seg = n // _SPLITS
    per_w = seg * l // _N_WORKERS
    chunk = 400
    while per_w % chunk or (per_w // chunk) % 2 == 0:
        chunk //= 2
    sub = chunk // 5

    outs = []
    for k in range(_SPLITS):
        sl = slice(k * seg, (k + 1) * seg)
        idx_flat = ids[sl].reshape(-1)
        xg = _sc_gather(features, idx_flat, chunk, sub)
        xg3 = xg.reshape(seg, l, _IN)
        outs.append(_tc_lstm_rgc(xg3, rels_f[sl], adj[sl], wih_t, whh_t, wflat))
    if _SPLITS == 1:
        return outs[0]
    return jnp.concatenate(outs, axis=0)


# speedup vs baseline: 1.3031x; 1.1052x over previous
import functools

import jax
import jax.numpy as jnp
from jax import lax
from jax.experimental import pallas as pl
from jax.experimental.pallas import tpu as pltpu
from jax.experimental.pallas import tpu_sc as plsc

_L = 32
_H = 128
_R = 11
_IN = 128
_OUT = 128

_N_WORKERS = 32

_SPLITS = 2

_BLK = 1000


def _sc_gather(table, idx_flat, chunk, sub):
    total = idx_flat.shape[0]
    d = table.shape[1]
    per_w = total // _N_WORKERS
    n_chunks = per_w // chunk
    n_pairs = (n_chunks - 1) // 2
    mesh = plsc.VectorSubcoreMesh(core_axis_name="c", subcore_axis_name="s")

    @functools.partial(
        pl.kernel,
        mesh=mesh,
        out_type=jax.ShapeDtypeStruct((total, d), jnp.float32),
        scratch_types=[
            pltpu.VMEM((per_w,), jnp.int32),
            pltpu.VMEM((chunk, d), jnp.float32),
            pltpu.VMEM((chunk, d), jnp.float32),
            pltpu.SemaphoreType.DMA,
            pltpu.SemaphoreType.DMA,
            pltpu.SemaphoreType.DMA,
            pltpu.SemaphoreType.DMA,
        ],
    )
    def gather_kernel(table_hbm, idx_hbm, out_hbm, idx_all, rows0, rows1,
                      sg0, sg1, ss0, ss1):
        wid = lax.axis_index("s") * 2 + lax.axis_index("c")
        base = wid * per_w
        pltpu.sync_copy(idx_hbm.at[pl.ds(base, per_w)], idx_all)

        def fire_gather(c, rows_buf, sem):
            for k in range(chunk // sub):
                pltpu.async_copy(
                    table_hbm.at[idx_all.at[pl.ds(c * chunk + k * sub, sub)]],
                    rows_buf.at[pl.ds(k * sub, sub)],
                    sem,
                )

        def drain_gather(rows_buf, sem):
            pltpu.make_async_copy(out_hbm.at[pl.ds(base, chunk)], rows_buf, sem).wait()

        def fire_store(c, rows_buf, sem):
            pltpu.async_copy(rows_buf, out_hbm.at[pl.ds(base + c * chunk, chunk)], sem)

        def drain_store(rows_buf, sem):
            pltpu.make_async_copy(rows_buf, out_hbm.at[pl.ds(base, chunk)], sem).wait()

        fire_gather(0, rows0, sg0)

        def body(p, carry):
            a = 2 * p + 1
            b = 2 * p + 2

            @pl.when(p > 0)
            def _():
                drain_store(rows1, ss1)
            fire_gather(a, rows1, sg1)
            drain_gather(rows0, sg0)
            fire_store(2 * p, rows0, ss0)
            drain_store(rows0, ss0)
            fire_gather(b, rows0, sg0)
            drain_gather(rows1, sg1)
            fire_store(a, rows1, ss1)
            return carry

        lax.fori_loop(0, n_pairs, body, 0)

        drain_gather(rows0, sg0)
        fire_store(n_chunks - 1, rows0, ss0)
        drain_store(rows0, ss0)
        drain_store(rows1, ss1)

    return gather_kernel(table, idx_flat)


_U = 8


def _tc_body(xg_ref, rels_ref, adj_ref, wih_ref, whh_ref, wflat_ref, out_ref,
             h_ref, c_ref, s_ref):
    tb = pl.program_id(1)

    @pl.when(tb == 0)
    def _init():
        h_ref[...] = jnp.zeros_like(h_ref)
        c_ref[...] = jnp.zeros_like(c_ref)
        s_ref[...] = jnp.zeros_like(s_ref)

    blk = adj_ref.shape[0]
    row = lax.broadcasted_iota(jnp.int32, (_L, 128), 0)
    col = lax.broadcasted_iota(jnp.int32, (_L, 128), 1)
    onehot = jnp.where((row == tb * _U + col) & (col < _U), 1.0, 0.0)
    cols_rel = jnp.dot(rels_ref[...], onehot, preferred_element_type=jnp.float32)
    adj_all = adj_ref[...]
    adj_hi = adj_all.astype(jnp.bfloat16).astype(jnp.float32)
    cols_adj = (jnp.dot(adj_hi, onehot, preferred_element_type=jnp.float32)
                + jnp.dot(adj_all - adj_hi, onehot, preferred_element_type=jnp.float32))

    h = h_ref[...]
    c = c_ref[...]
    for u in range(_U):
        x_u = xg_ref[:, u, :]
        gates = jnp.dot(x_u, wih_ref[...], preferred_element_type=jnp.float32)
        gates = gates + jnp.dot(h, whh_ref[...], preferred_element_type=jnp.float32)
        i_g = jax.nn.sigmoid(gates[:, 0:_H])
        f_g = jax.nn.sigmoid(gates[:, _H:2 * _H])
        g_g = jnp.tanh(gates[:, 2 * _H:3 * _H])
        o_g = jax.nn.sigmoid(gates[:, 3 * _H:4 * _H])
        c = f_g * c + i_g * g_g
        h = o_g * jnp.tanh(c)
        rel_u = jnp.broadcast_to(cols_rel[:, u:u + 1], (blk, _H))
        adj_u = jnp.broadcast_to(cols_adj[:, u:u + 1], (blk, _H))
        w_u = adj_u * h
        for r in range(_R):
            s_ref[:, r * _H:(r + 1) * _H] += jnp.where(rel_u == r, w_u, 0.0)
    h_ref[...] = h
    c_ref[...] = c

    @pl.when(tb == _L // _U - 1)
    def _finish():
        out_ref[...] = jnp.maximum(
            jnp.dot(s_ref[...], wflat_ref[...], preferred_element_type=jnp.float32),
            0.0,
        )


def _tc_lstm_rgc(xg3, rels, adj, wih_t, whh_t, wflat, interpret=False):
    n = rels.shape[0]
    grid = (n // _BLK, _L // _U)
    return pl.pallas_call(
        _tc_body,
        grid=grid,
        in_specs=[
            pl.BlockSpec((_BLK, _U, _IN), lambda i, t: (i, t, 0)),
            pl.BlockSpec((_BLK, _L), lambda i, t: (i, 0)),
            pl.BlockSpec((_BLK, _L), lambda i, t: (i, 0)),
            pl.BlockSpec((_IN, 4 * _H), lambda i, t: (0, 0)),
            pl.BlockSpec((_H, 4 * _H), lambda i, t: (0, 0)),
            pl.BlockSpec((_R * _H, _OUT), lambda i, t: (0, 0)),
        ],
        out_specs=pl.BlockSpec((_BLK, _OUT), lambda i, t: (i, 0)),
        out_shape=jax.ShapeDtypeStruct((n, _OUT), jnp.float32),
        scratch_shapes=[
            pltpu.VMEM((_BLK, _H), jnp.float32),
            pltpu.VMEM((_BLK, _H), jnp.float32),
            pltpu.VMEM((_BLK, _R * _H), jnp.float32),
        ],
        interpret=interpret,
    )(xg3, rels, adj, wih_t, whh_t, wflat)


def kernel(features, neighbor_ids, neighbor_rels, adj_weights, W_ih, W_hh, rgc_weights):
    n, l = neighbor_ids.shape
    ids = neighbor_ids.astype(jnp.int32)
    rels_f = neighbor_rels.astype(jnp.float32)
    adj = adj_weights.astype(jnp.float32)
    wih_t = W_ih.T
    whh_t = W_hh.T
    wflat = rgc_weights.reshape(_R * _H, _OUT)

    seg = n // _SPLITS
    per_w = seg * l // _N_WORKERS
    chunk = 400
    while per_w % chunk or (per_w // chunk) % 2 == 0:
        chunk //= 2
    sub = chunk // 5

    outs = []
    for k in range(_SPLITS):
        sl = slice(k * seg, (k + 1) * seg)
        idx_flat = ids[sl].reshape(-1)
        xg = _sc_gather(features, idx_flat, chunk, sub)
        xg3 = xg.reshape(seg, l, _IN)
        outs.append(_tc_lstm_rgc(xg3, rels_f[sl], adj[sl], wih_t, whh_t, wflat))
    if _SPLITS == 1:
        return outs[0]
    return jnp.concatenate(outs, axis=0)
